# baseline (speedup 1.0000x reference)
import functools

import numpy as np
import jax
import jax.numpy as jnp
from jax import lax
from jax.experimental import pallas as pl
from jax.experimental.pallas import tpu as pltpu
from jax.experimental.pallas import tpu_sc as plsc

_N = 10000
_E = 320000
_D = 128
_R2 = 200
_NPAD = 10240
_EC = 3200
_NC_CH = 1000
_SC_CH = 80


def _dft_consts():
    n = np.arange(_D)
    k = np.arange(65)
    ang_nk = 2.0 * np.pi * np.outer(n, k) / _D
    cos_nk = np.cos(ang_nk)
    sin_nk = np.sin(ang_nk)
    dft2 = np.zeros((_D, 2 * _D), np.float64)
    dft2[:, :65] = cos_nk
    dft2[:, 65:128] = -sin_nk[:, 1:64]
    dft2[:, 128:193] = -sin_nk
    dft2[:, 193:] = cos_nk[:, 1:64]
    ang_kn = ang_nk.T
    c = np.where((k == 0) | (k == 64), 1.0, 2.0)[:, None] / _D
    ga = np.zeros((_D, _D), np.float64)
    ga[:65] = c * np.cos(ang_kn)
    ga[65:] = (2.0 / _D) * np.cos(ang_kn[1:64])
    gb = np.zeros((_D, _D), np.float64)
    gb[:65] = -(2.0 / _D) * np.sin(ang_kn)
    gb[65:] = (2.0 / _D) * np.sin(ang_kn[1:64])
    return (dft2.astype(np.float32), ga.astype(np.float32),
            gb.astype(np.float32))


_DFT2, _GA, _GB = _dft_consts()


def _prep_body(x_ref, rel_ref, lr_ref, win_ref, wout_ref, wl_ref, wr_ref,
               dft2_ref, ga_ref, gb_ref,
               fx_ref, frel2_ref, wmsg_ref, wle_ref, relout_ref):
    f32 = jnp.float32
    fx_ref[...] = jnp.dot(x_ref[...], dft2_ref[:, :_D],
                          preferred_element_type=f32)

    @pl.when(pl.program_id(0) == 0)
    def _():
        ga = ga_ref[...]
        gb = gb_ref[...]
        frel2_ref[...] = jnp.dot(
            rel_ref[...], dft2_ref[...],
            preferred_element_type=f32).astype(jnp.bfloat16)
        wmsg_ref[0, :_D] = jnp.dot(
            ga, win_ref[...], preferred_element_type=f32).astype(jnp.bfloat16)
        wmsg_ref[0, _D:] = jnp.dot(
            gb, win_ref[...], preferred_element_type=f32).astype(jnp.bfloat16)
        wmsg_ref[1, :_D] = jnp.dot(
            ga, wout_ref[...],
            preferred_element_type=f32).astype(jnp.bfloat16)
        wmsg_ref[1, _D:] = jnp.dot(
            gb, wout_ref[...],
            preferred_element_type=f32).astype(jnp.bfloat16)
        bl2 = jnp.dot(lr_ref[...], dft2_ref[...], preferred_element_type=f32)
        gaw = jnp.dot(ga, wl_ref[...], preferred_element_type=f32)
        gbw = jnp.dot(gb, wl_ref[...], preferred_element_type=f32)
        wle_ref[...] = (bl2[0, :_D, None] * gaw + bl2[0, _D:, None] * gbw)
        relout_ref[...] = jnp.dot(rel_ref[...], wr_ref[...],
                                  preferred_element_type=f32)


def _tc_prep(x, rel, loop_rel, w_in, w_out, w_loop, w_rel, dft2, ga, gb):
    zero2 = lambda i: (0, 0)
    zero3 = lambda i: (0, 0, 0)
    return pl.pallas_call(
        _prep_body,
        grid=(_N // _NC_CH,),
        in_specs=[
            pl.BlockSpec((_NC_CH, _D), lambda i: (i, 0)),
            pl.BlockSpec((_R2, _D), zero2),
            pl.BlockSpec((1, _D), zero2),
            pl.BlockSpec((_D, _D), zero2),
            pl.BlockSpec((_D, _D), zero2),
            pl.BlockSpec((_D, _D), zero2),
            pl.BlockSpec((_D, _D), zero2),
            pl.BlockSpec((_D, 2 * _D), zero2),
            pl.BlockSpec((_D, _D), zero2),
            pl.BlockSpec((_D, _D), zero2),
        ],
        out_specs=[
            pl.BlockSpec((_NC_CH, _D), lambda i: (i, 0)),
            pl.BlockSpec((_R2, 2 * _D), zero2),
            pl.BlockSpec((2, 2 * _D, _D), zero3),
            pl.BlockSpec((_D, _D), zero2),
            pl.BlockSpec((_R2, _D), zero2),
        ],
        out_shape=[
            jax.ShapeDtypeStruct((_N, _D), jnp.float32),
            jax.ShapeDtypeStruct((_R2, 2 * _D), jnp.bfloat16),
            jax.ShapeDtypeStruct((2, 2 * _D, _D), jnp.bfloat16),
            jax.ShapeDtypeStruct((_D, _D), jnp.float32),
            jax.ShapeDtypeStruct((_R2, _D), jnp.float32),
        ],
    )(x, rel, loop_rel, w_in, w_out, w_loop, w_rel, dft2, ga, gb)


def _sc_gather(table, idx3d, e, ch):
    mesh = plsc.VectorSubcoreMesh(core_axis_name="c", subcore_axis_name="s")
    nc, ns = mesh.num_cores, mesh.num_subcores
    per_w = e // (nc * ns)
    iters = per_w // ch

    assert iters % 2 == 1 and iters >= 3
    pairs = (iters - 1) // 2

    @functools.partial(
        pl.kernel,
        out_type=jax.ShapeDtypeStruct((e, _D), jnp.float32),
        mesh=mesh,
        scratch_types=[
            pltpu.VMEM((iters, ch), jnp.int32),
            pltpu.VMEM((ch, _D), jnp.float32),
            pltpu.VMEM((ch, _D), jnp.float32),
            pltpu.SemaphoreType.DMA,
            pltpu.SemaphoreType.DMA,
        ],
    )
    def gk(table_hbm, idx_hbm, out_hbm, idx_v, rows0, rows1, sem0, sem1):
        c = lax.axis_index("c")
        s = lax.axis_index("s")
        wid = s * nc + c
        base = wid * per_w
        pltpu.sync_copy(idx_hbm.at[wid], idx_v)

        def fire(i, rows, sem):
            pltpu.async_copy(table_hbm.at[idx_v.at[i]], rows, sem)

        def drain(i, rows, sem):
            pltpu.make_async_copy(table_hbm.at[idx_v.at[i]], rows,
                                  sem).wait()
            pltpu.sync_copy(rows,
                            out_hbm.at[pl.ds(base + i * ch, ch)])

        fire(0, rows0, sem0)

        def body(j, carry):
            fire(2 * j + 1, rows1, sem1)
            drain(2 * j, rows0, sem0)
            fire(2 * j + 2, rows0, sem0)
            drain(2 * j + 1, rows1, sem1)
            return carry

        lax.fori_loop(0, pairs, body, 0)
        drain(iters - 1, rows0, sem0)

    return gk(table, idx3d)


def _msg_body(z_ref, et_ref, nrm_ref, frel_ref, w_ref, o_ref):
    f32 = jnp.float32
    et_row = et_ref[0]
    nrm_row = nrm_ref[0]
    iota_r = lax.broadcasted_iota(jnp.int32, (_R2, _EC), 0)
    et_b = jnp.broadcast_to(et_row, (_R2, _EC))
    nrm_b = jnp.broadcast_to(nrm_row, (_R2, _EC))
    oh_t = jnp.where(et_b == iota_r, nrm_b,
                     jnp.float32(0)).astype(jnp.bfloat16)
    b2 = lax.dot_general(oh_t, frel_ref[...],
                         (((0,), (0,)), ((), ())),
                         preferred_element_type=f32)
    a = z_ref[...]
    p = (a * b2[:, :_D]).astype(jnp.bfloat16)
    q = (a * b2[:, _D:]).astype(jnp.bfloat16)
    pq = jnp.concatenate([p, q], axis=1)
    o_ref[...] = jnp.dot(pq, w_ref[0], preferred_element_type=f32)


def _tc_msg(zsrc, edge_type3, norm3, frel2, wmsg, e, half):
    nblk = e // _EC
    return pl.pallas_call(
        _msg_body,
        grid=(nblk,),
        in_specs=[
            pl.BlockSpec((_EC, _D), lambda i: (i, 0)),
            pl.BlockSpec((1, 1, _EC), lambda i: (i, 0, 0)),
            pl.BlockSpec((1, 1, _EC), lambda i: (i, 0, 0)),
            pl.BlockSpec((_R2, 2 * _D), lambda i: (0, 0)),
            pl.BlockSpec((1, 2 * _D, _D), lambda i: (half, 0, 0)),
        ],
        out_specs=pl.BlockSpec((_EC, _D), lambda i: (i, 0)),
        out_shape=jax.ShapeDtypeStruct((e, _D), jnp.float32),
    )(zsrc, edge_type3, norm3, frel2, wmsg)


def _sc_scatter(msg, dst3d, e, ch):
    mesh = plsc.VectorSubcoreMesh(core_axis_name="c", subcore_axis_name="s")
    nc, ns = mesh.num_cores, mesh.num_subcores
    per_w = e // (nc * ns)
    iters = per_w // ch
    rows_s = _NPAD // ns
    zreps = rows_s // ch

    assert iters % 2 == 1 and iters >= 3
    pairs = (iters - 1) // 2

    @functools.partial(
        pl.kernel,
        out_type=jax.ShapeDtypeStruct((nc, _NPAD, _D), jnp.float32),
        mesh=mesh,
        scratch_types=[
            pltpu.VMEM((iters, ch), jnp.int32),
            pltpu.VMEM((ch, _D), jnp.float32),
            pltpu.VMEM((ch, _D), jnp.float32),
            pltpu.VMEM_SHARED((_NPAD, _D), jnp.float32),
            pltpu.SemaphoreType.DMA,
            pltpu.SemaphoreType.DMA,
        ],
    )
    def sk(msg_hbm, dst_hbm, out_hbm, idx_v, rows0, rows1, acc_sh,
           sem0, sem1):
        c = lax.axis_index("c")
        s = lax.axis_index("s")
        wid = s * nc + c
        base = wid * per_w
        pltpu.sync_copy(dst_hbm.at[wid], idx_v)

        def zb(i, carry):
            rows0[i // 8, pl.ds((i % 8) * 16, 16)] = jnp.zeros((16,),
                                                               jnp.float32)
            return carry

        lax.fori_loop(0, ch * 8, zb, 0)

        def zs(j, carry):
            pltpu.sync_copy(
                rows0,
                acc_sh.at[pl.ds(s * rows_s + j * ch, ch)])
            return carry

        lax.fori_loop(0, zreps, zs, 0)
        plsc.subcore_barrier()

        def fire(i, rows, sem):
            pltpu.async_copy(msg_hbm.at[pl.ds(base + i * ch, ch)],
                             rows, sem)

        def drain(i, rows, sem):
            pltpu.make_async_copy(msg_hbm.at[pl.ds(base + i * ch, ch)],
                                  rows, sem).wait()
            pltpu.sync_copy(rows, acc_sh.at[idx_v.at[i]], add=True)

        fire(0, rows0, sem0)

        def body(j, carry):
            fire(2 * j + 1, rows1, sem1)
            drain(2 * j, rows0, sem0)
            fire(2 * j + 2, rows0, sem0)
            drain(2 * j + 1, rows1, sem1)
            return carry

        lax.fori_loop(0, pairs, body, 0)
        drain(iters - 1, rows0, sem0)
        plsc.subcore_barrier()
        pltpu.sync_copy(acc_sh.at[pl.ds(s * rows_s, rows_s)],
                        out_hbm.at[c, pl.ds(s * rows_s, rows_s)])

    return sk(msg, dst3d)


_NB = _N // _NC_CH


def _fin_body(p_ref, p2_ref, fx_ref, wl_ref, b_ref, g_ref, bt_ref, o_ref,
              t_ref, st_ref):
    i = pl.program_id(0)

    @pl.when(i < _NB)
    def _():
        lm = jnp.dot(fx_ref[...], wl_ref[...],
                     preferred_element_type=jnp.float32)
        tt = ((p_ref[0] + p_ref[1] + p2_ref[0] + p2_ref[1] + lm)
              * jnp.float32(1.0 / 3.0) + b_ref[...])
        blk = lax.rem(i, _NB)
        t_ref[pl.ds(blk * _NC_CH, _NC_CH), :] = tt
        s1 = jnp.sum(tt, axis=0, keepdims=True)
        s2 = jnp.sum(tt * tt, axis=0, keepdims=True)
        upd = jnp.concatenate([s1, s2, jnp.zeros((6, _D), jnp.float32)],
                              axis=0)

        @pl.when(i == 0)
        def _():
            st_ref[...] = jnp.zeros((8, _D), jnp.float32)

        st_ref[...] += upd

    @pl.when(i >= _NB)
    def _():
        inv_n = jnp.float32(1.0 / _N)
        mean = st_ref[0:1] * inv_n
        var = st_ref[1:2] * inv_n - mean * mean
        inv = lax.rsqrt(var + jnp.float32(1e-5))
        blk = lax.rem(i, _NB)
        tt = t_ref[pl.ds(blk * _NC_CH, _NC_CH), :]
        o_ref[...] = jnp.tanh((tt - mean) * inv * g_ref[...] + bt_ref[...])


def _tc_finish(partials, partials2, fx, wloop_eff, bias, gamma, beta):
    mod = lambda i: (lax.rem(i, _NB), 0)
    zero2 = lambda i: (0, 0)
    return pl.pallas_call(
        _fin_body,
        grid=(2 * _NB,),
        in_specs=[
            pl.BlockSpec((2, _NC_CH, _D), lambda i: (0, lax.rem(i, _NB), 0)),
            pl.BlockSpec((2, _NC_CH, _D), lambda i: (0, lax.rem(i, _NB), 0)),
            pl.BlockSpec((_NC_CH, _D), mod),
            pl.BlockSpec((_D, _D), zero2),
            pl.BlockSpec((1, _D), zero2),
            pl.BlockSpec((1, _D), zero2),
            pl.BlockSpec((1, _D), zero2),
        ],
        out_specs=pl.BlockSpec((_NC_CH, _D), mod),
        out_shape=jax.ShapeDtypeStruct((_N, _D), jnp.float32),
        scratch_shapes=[
            pltpu.VMEM((_N, _D), jnp.float32),
            pltpu.VMEM((8, _D), jnp.float32),
        ],
    )(partials, partials2, fx, wloop_eff, bias, gamma, beta)


def kernel(x, edge_index, edge_type, norm, rel_embeds, w_loop, w_in, w_out,
           w_rel, loop_rel, w_bias, bn_gamma, bn_beta):
    src = edge_index[0]
    dst = edge_index[1]
    dft2 = jnp.asarray(_DFT2)
    dftc = dft2[:, :_D]
    ga = jnp.asarray(_GA)
    gb = jnp.asarray(_GB)

    fx, frel2, wmsg, wloop_eff, rel_out = _tc_prep(
        x, rel_embeds, loop_rel, w_in, w_out, w_loop, w_rel, dft2, ga, gb)
    nw = _E // (_SC_CH * 125)
    eh = _E // 2
    ch = _SC_CH // 2
    nblk_h = eh // _EC
    parts = []
    for h in (0, 1):
        sl = slice(h * eh, (h + 1) * eh)
        zsrc = _sc_gather(fx, src[sl].reshape(nw, 125, ch), eh, ch)
        msg = _tc_msg(zsrc, edge_type[sl].reshape(nblk_h, 1, _EC),
                      norm[sl].reshape(nblk_h, 1, _EC), frel2, wmsg,
                      eh, h)
        parts.append(_sc_scatter(msg, dst[sl].reshape(nw, 125, ch), eh, ch))
    node_repr = _tc_finish(parts[0], parts[1], fx, wloop_eff,
                           w_bias.reshape(1, _D),
                           bn_gamma.reshape(1, _D), bn_beta.reshape(1, _D))
    return node_repr, rel_out


# speedup vs baseline: 1.0731x; 1.0731x over previous
import functools

import numpy as np
import jax
import jax.numpy as jnp
from jax import lax
from jax.experimental import pallas as pl
from jax.experimental.pallas import tpu as pltpu
from jax.experimental.pallas import tpu_sc as plsc

_N = 10000
_E = 320000
_D = 128
_R2 = 200
_NPAD = 10240
_EC = 6400
_NC_CH = 1000
_SC_CH = 80


def _dft_consts():
    n = np.arange(_D)
    k = np.arange(65)
    ang_nk = 2.0 * np.pi * np.outer(n, k) / _D
    cos_nk = np.cos(ang_nk)
    sin_nk = np.sin(ang_nk)
    dft2 = np.zeros((_D, 2 * _D), np.float64)
    dft2[:, :65] = cos_nk
    dft2[:, 65:128] = -sin_nk[:, 1:64]
    dft2[:, 128:193] = -sin_nk
    dft2[:, 193:] = cos_nk[:, 1:64]
    ang_kn = ang_nk.T
    c = np.where((k == 0) | (k == 64), 1.0, 2.0)[:, None] / _D
    ga = np.zeros((_D, _D), np.float64)
    ga[:65] = c * np.cos(ang_kn)
    ga[65:] = (2.0 / _D) * np.cos(ang_kn[1:64])
    gb = np.zeros((_D, _D), np.float64)
    gb[:65] = -(2.0 / _D) * np.sin(ang_kn)
    gb[65:] = (2.0 / _D) * np.sin(ang_kn[1:64])
    return (dft2.astype(np.float32), ga.astype(np.float32),
            gb.astype(np.float32))


_DFT2, _GA, _GB = _dft_consts()


def _prep_body(x_ref, rel_ref, lr_ref, win_ref, wout_ref, wl_ref, wr_ref,
               dft2_ref, ga_ref, gb_ref,
               fx_ref, frel2_ref, wmsg_ref, wle_ref, relout_ref):
    f32 = jnp.float32
    fx_ref[...] = jnp.dot(x_ref[...], dft2_ref[:, :_D],
                          preferred_element_type=f32)

    @pl.when(pl.program_id(0) == 0)
    def _():
        ga = ga_ref[...]
        gb = gb_ref[...]
        frel2_ref[...] = jnp.dot(
            rel_ref[...], dft2_ref[...],
            preferred_element_type=f32).astype(jnp.bfloat16)
        wmsg_ref[0, :_D] = jnp.dot(
            ga, win_ref[...], preferred_element_type=f32).astype(jnp.bfloat16)
        wmsg_ref[0, _D:] = jnp.dot(
            gb, win_ref[...], preferred_element_type=f32).astype(jnp.bfloat16)
        wmsg_ref[1, :_D] = jnp.dot(
            ga, wout_ref[...],
            preferred_element_type=f32).astype(jnp.bfloat16)
        wmsg_ref[1, _D:] = jnp.dot(
            gb, wout_ref[...],
            preferred_element_type=f32).astype(jnp.bfloat16)
        bl2 = jnp.dot(lr_ref[...], dft2_ref[...], preferred_element_type=f32)
        gaw = jnp.dot(ga, wl_ref[...], preferred_element_type=f32)
        gbw = jnp.dot(gb, wl_ref[...], preferred_element_type=f32)
        wle_ref[...] = (bl2[0, :_D, None] * gaw + bl2[0, _D:, None] * gbw)
        relout_ref[...] = jnp.dot(rel_ref[...], wr_ref[...],
                                  preferred_element_type=f32)


def _tc_prep(x, rel, loop_rel, w_in, w_out, w_loop, w_rel, dft2, ga, gb):
    zero2 = lambda i: (0, 0)
    zero3 = lambda i: (0, 0, 0)
    return pl.pallas_call(
        _prep_body,
        grid=(_N // _NC_CH,),
        in_specs=[
            pl.BlockSpec((_NC_CH, _D), lambda i: (i, 0)),
            pl.BlockSpec((_R2, _D), zero2),
            pl.BlockSpec((1, _D), zero2),
            pl.BlockSpec((_D, _D), zero2),
            pl.BlockSpec((_D, _D), zero2),
            pl.BlockSpec((_D, _D), zero2),
            pl.BlockSpec((_D, _D), zero2),
            pl.BlockSpec((_D, 2 * _D), zero2),
            pl.BlockSpec((_D, _D), zero2),
            pl.BlockSpec((_D, _D), zero2),
        ],
        out_specs=[
            pl.BlockSpec((_NC_CH, _D), lambda i: (i, 0)),
            pl.BlockSpec((_R2, 2 * _D), zero2),
            pl.BlockSpec((2, 2 * _D, _D), zero3),
            pl.BlockSpec((_D, _D), zero2),
            pl.BlockSpec((_R2, _D), zero2),
        ],
        out_shape=[
            jax.ShapeDtypeStruct((_N, _D), jnp.float32),
            jax.ShapeDtypeStruct((_R2, 2 * _D), jnp.bfloat16),
            jax.ShapeDtypeStruct((2, 2 * _D, _D), jnp.bfloat16),
            jax.ShapeDtypeStruct((_D, _D), jnp.float32),
            jax.ShapeDtypeStruct((_R2, _D), jnp.float32),
        ],
    )(x, rel, loop_rel, w_in, w_out, w_loop, w_rel, dft2, ga, gb)


def _sc_gather(table, idx2d):
    mesh = plsc.VectorSubcoreMesh(core_axis_name="c", subcore_axis_name="s")
    nc, ns = mesh.num_cores, mesh.num_subcores
    per_w = _E // (nc * ns)
    iters = per_w // _SC_CH

    assert iters % 2 == 1 and iters >= 3
    pairs = (iters - 1) // 2

    @functools.partial(
        pl.kernel,
        out_type=jax.ShapeDtypeStruct((_E, _D), jnp.float32),
        mesh=mesh,
        scratch_types=[
            pltpu.VMEM((iters, _SC_CH), jnp.int32),
            pltpu.VMEM((_SC_CH, _D), jnp.float32),
            pltpu.VMEM((_SC_CH, _D), jnp.float32),
            pltpu.SemaphoreType.DMA,
            pltpu.SemaphoreType.DMA,
        ],
    )
    def gk(table_hbm, idx_hbm, out_hbm, idx_v, rows0, rows1, sem0, sem1):
        c = lax.axis_index("c")
        s = lax.axis_index("s")
        wid = s * nc + c
        base = wid * per_w
        pltpu.sync_copy(idx_hbm.at[wid], idx_v)

        def fire(i, rows, sem):
            pltpu.async_copy(table_hbm.at[idx_v.at[i]], rows, sem)

        def drain(i, rows, sem):
            pltpu.make_async_copy(table_hbm.at[idx_v.at[i]], rows,
                                  sem).wait()
            pltpu.sync_copy(rows,
                            out_hbm.at[pl.ds(base + i * _SC_CH, _SC_CH)])

        fire(0, rows0, sem0)

        def body(j, carry):
            fire(2 * j + 1, rows1, sem1)
            drain(2 * j, rows0, sem0)
            fire(2 * j + 2, rows0, sem0)
            drain(2 * j + 1, rows1, sem1)
            return carry

        lax.fori_loop(0, pairs, body, 0)
        drain(iters - 1, rows0, sem0)

    return gk(table, idx2d)


def _msg_body(z_ref, et_ref, nrm_ref, frel_ref, w_ref, o_ref):
    f32 = jnp.float32
    et_row = et_ref[0]
    nrm_row = nrm_ref[0]
    iota_r = lax.broadcasted_iota(jnp.int32, (_R2, _EC), 0)
    et_b = jnp.broadcast_to(et_row, (_R2, _EC))
    nrm_b = jnp.broadcast_to(nrm_row, (_R2, _EC))
    oh_t = jnp.where(et_b == iota_r, nrm_b,
                     jnp.float32(0)).astype(jnp.bfloat16)
    b2 = lax.dot_general(oh_t, frel_ref[...],
                         (((0,), (0,)), ((), ())),
                         preferred_element_type=f32)
    a = z_ref[...]
    p = (a * b2[:, :_D]).astype(jnp.bfloat16)
    q = (a * b2[:, _D:]).astype(jnp.bfloat16)
    pq = jnp.concatenate([p, q], axis=1)
    o_ref[...] = jnp.dot(pq, w_ref[0], preferred_element_type=f32)


def _tc_msg(zsrc, edge_type3, norm3, frel2, wmsg):
    nblk = _E // _EC
    half_blk = nblk // 2
    return pl.pallas_call(
        _msg_body,
        grid=(nblk,),
        in_specs=[
            pl.BlockSpec((_EC, _D), lambda i: (i, 0)),
            pl.BlockSpec((1, 1, _EC), lambda i: (i, 0, 0)),
            pl.BlockSpec((1, 1, _EC), lambda i: (i, 0, 0)),
            pl.BlockSpec((_R2, 2 * _D), lambda i: (0, 0)),
            pl.BlockSpec((1, 2 * _D, _D), lambda i: (i // half_blk, 0, 0)),
        ],
        out_specs=pl.BlockSpec((_EC, _D), lambda i: (i, 0)),
        out_shape=jax.ShapeDtypeStruct((_E, _D), jnp.float32),
    )(zsrc, edge_type3, norm3, frel2, wmsg)


def _sc_scatter(msg, dst):
    mesh = plsc.VectorSubcoreMesh(core_axis_name="c", subcore_axis_name="s")
    nc, ns = mesh.num_cores, mesh.num_subcores
    per_w = _E // (nc * ns)
    iters = per_w // _SC_CH
    rows_s = _NPAD // ns
    zreps = rows_s // _SC_CH

    assert iters % 2 == 1 and iters >= 3
    pairs = (iters - 1) // 2

    @functools.partial(
        pl.kernel,
        out_type=jax.ShapeDtypeStruct((nc, _NPAD, _D), jnp.float32),
        mesh=mesh,
        scratch_types=[
            pltpu.VMEM((iters, _SC_CH), jnp.int32),
            pltpu.VMEM((_SC_CH, _D), jnp.float32),
            pltpu.VMEM((_SC_CH, _D), jnp.float32),
            pltpu.VMEM_SHARED((_NPAD, _D), jnp.float32),
            pltpu.SemaphoreType.DMA,
            pltpu.SemaphoreType.DMA,
        ],
    )
    def sk(msg_hbm, dst_hbm, out_hbm, idx_v, rows0, rows1, acc_sh,
           sem0, sem1):
        c = lax.axis_index("c")
        s = lax.axis_index("s")
        wid = s * nc + c
        base = wid * per_w
        pltpu.sync_copy(dst_hbm.at[wid], idx_v)

        def zb(i, carry):
            rows0[i // 8, pl.ds((i % 8) * 16, 16)] = jnp.zeros((16,),
                                                               jnp.float32)
            return carry

        lax.fori_loop(0, _SC_CH * 8, zb, 0)

        def zs(j, carry):
            pltpu.sync_copy(
                rows0,
                acc_sh.at[pl.ds(s * rows_s + j * _SC_CH, _SC_CH)])
            return carry

        lax.fori_loop(0, zreps, zs, 0)
        plsc.subcore_barrier()

        def fire(i, rows, sem):
            pltpu.async_copy(msg_hbm.at[pl.ds(base + i * _SC_CH, _SC_CH)],
                             rows, sem)

        def drain(i, rows, sem):
            pltpu.make_async_copy(msg_hbm.at[pl.ds(base + i * _SC_CH,
                                                   _SC_CH)], rows,
                                  sem).wait()
            pltpu.sync_copy(rows, acc_sh.at[idx_v.at[i]], add=True)

        fire(0, rows0, sem0)

        def body(j, carry):
            fire(2 * j + 1, rows1, sem1)
            drain(2 * j, rows0, sem0)
            fire(2 * j + 2, rows0, sem0)
            drain(2 * j + 1, rows1, sem1)
            return carry

        lax.fori_loop(0, pairs, body, 0)
        drain(iters - 1, rows0, sem0)
        plsc.subcore_barrier()
        pltpu.sync_copy(acc_sh.at[pl.ds(s * rows_s, rows_s)],
                        out_hbm.at[c, pl.ds(s * rows_s, rows_s)])

    return sk(msg, dst)


_NB = _N // _NC_CH


def _fin_body(p_ref, fx_ref, wl_ref, b_ref, g_ref, bt_ref, o_ref,
              t_ref, st_ref):
    i = pl.program_id(0)

    @pl.when(i < _NB)
    def _():
        lm = jnp.dot(fx_ref[...], wl_ref[...],
                     preferred_element_type=jnp.float32)
        tt = ((p_ref[0] + p_ref[1] + lm) * jnp.float32(1.0 / 3.0)
              + b_ref[...])
        blk = lax.rem(i, _NB)
        t_ref[pl.ds(blk * _NC_CH, _NC_CH), :] = tt
        s1 = jnp.sum(tt, axis=0, keepdims=True)
        s2 = jnp.sum(tt * tt, axis=0, keepdims=True)
        upd = jnp.concatenate([s1, s2, jnp.zeros((6, _D), jnp.float32)],
                              axis=0)

        @pl.when(i == 0)
        def _():
            st_ref[...] = jnp.zeros((8, _D), jnp.float32)

        st_ref[...] += upd

    @pl.when(i >= _NB)
    def _():
        inv_n = jnp.float32(1.0 / _N)
        mean = st_ref[0:1] * inv_n
        var = st_ref[1:2] * inv_n - mean * mean
        inv = lax.rsqrt(var + jnp.float32(1e-5))
        blk = lax.rem(i, _NB)
        tt = t_ref[pl.ds(blk * _NC_CH, _NC_CH), :]
        o_ref[...] = jnp.tanh((tt - mean) * inv * g_ref[...] + bt_ref[...])


def _tc_finish(partials, fx, wloop_eff, bias, gamma, beta):
    mod = lambda i: (lax.rem(i, _NB), 0)
    zero2 = lambda i: (0, 0)
    return pl.pallas_call(
        _fin_body,
        grid=(2 * _NB,),
        in_specs=[
            pl.BlockSpec((2, _NC_CH, _D), lambda i: (0, lax.rem(i, _NB), 0)),
            pl.BlockSpec((_NC_CH, _D), mod),
            pl.BlockSpec((_D, _D), zero2),
            pl.BlockSpec((1, _D), zero2),
            pl.BlockSpec((1, _D), zero2),
            pl.BlockSpec((1, _D), zero2),
        ],
        out_specs=pl.BlockSpec((_NC_CH, _D), mod),
        out_shape=jax.ShapeDtypeStruct((_N, _D), jnp.float32),
        scratch_shapes=[
            pltpu.VMEM((_N, _D), jnp.float32),
            pltpu.VMEM((8, _D), jnp.float32),
        ],
    )(partials, fx, wloop_eff, bias, gamma, beta)


def kernel(x, edge_index, edge_type, norm, rel_embeds, w_loop, w_in, w_out,
           w_rel, loop_rel, w_bias, bn_gamma, bn_beta):
    src = edge_index[0]
    dst = edge_index[1]
    dft2 = jnp.asarray(_DFT2)
    dftc = dft2[:, :_D]
    ga = jnp.asarray(_GA)
    gb = jnp.asarray(_GB)

    fx, frel2, wmsg, wloop_eff, rel_out = _tc_prep(
        x, rel_embeds, loop_rel, w_in, w_out, w_loop, w_rel, dft2, ga, gb)
    nw = _E // (_SC_CH * 125)
    zsrc = _sc_gather(fx, src.reshape(nw, 125, _SC_CH))
    nblk = _E // _EC
    msg = _tc_msg(zsrc, edge_type.reshape(nblk, 1, _EC),
                  norm.reshape(nblk, 1, _EC), frel2, wmsg)
    partials = _sc_scatter(msg, dst.reshape(nw, 125, _SC_CH))
    node_repr = _tc_finish(partials, fx, wloop_eff, w_bias.reshape(1, _D),
                           bn_gamma.reshape(1, _D), bn_beta.reshape(1, _D))
    return node_repr, rel_out


# speedup vs baseline: 1.0911x; 1.0168x over previous
import functools

import numpy as np
import jax
import jax.numpy as jnp
from jax import lax
from jax.experimental import pallas as pl
from jax.experimental.pallas import tpu as pltpu
from jax.experimental.pallas import tpu_sc as plsc

_N = 10000
_E = 320000
_D = 128
_R2 = 200
_NPAD = 10240
_EC = 8000
_NC_CH = 1000
_SC_CH = 80


def _dft_consts():
    n = np.arange(_D)
    k = np.arange(65)
    ang_nk = 2.0 * np.pi * np.outer(n, k) / _D
    cos_nk = np.cos(ang_nk)
    sin_nk = np.sin(ang_nk)
    dft2 = np.zeros((_D, 2 * _D), np.float64)
    dft2[:, :65] = cos_nk
    dft2[:, 65:128] = -sin_nk[:, 1:64]
    dft2[:, 128:193] = -sin_nk
    dft2[:, 193:] = cos_nk[:, 1:64]
    ang_kn = ang_nk.T
    c = np.where((k == 0) | (k == 64), 1.0, 2.0)[:, None] / _D
    ga = np.zeros((_D, _D), np.float64)
    ga[:65] = c * np.cos(ang_kn)
    ga[65:] = (2.0 / _D) * np.cos(ang_kn[1:64])
    gb = np.zeros((_D, _D), np.float64)
    gb[:65] = -(2.0 / _D) * np.sin(ang_kn)
    gb[65:] = (2.0 / _D) * np.sin(ang_kn[1:64])
    return (dft2.astype(np.float32), ga.astype(np.float32),
            gb.astype(np.float32))


_DFT2, _GA, _GB = _dft_consts()


def _prep_body(x_ref, rel_ref, lr_ref, win_ref, wout_ref, wl_ref, wr_ref,
               dft2_ref, ga_ref, gb_ref,
               fx_ref, frel2_ref, wmsg_ref, wle_ref, relout_ref):
    f32 = jnp.float32
    fx_ref[...] = jnp.dot(x_ref[...], dft2_ref[:, :_D],
                          preferred_element_type=f32)

    @pl.when(pl.program_id(0) == 0)
    def _():
        ga = ga_ref[...]
        gb = gb_ref[...]
        frel2_ref[...] = jnp.dot(
            rel_ref[...], dft2_ref[...],
            preferred_element_type=f32).astype(jnp.bfloat16)
        wmsg_ref[0, :_D] = jnp.dot(
            ga, win_ref[...], preferred_element_type=f32).astype(jnp.bfloat16)
        wmsg_ref[0, _D:] = jnp.dot(
            gb, win_ref[...], preferred_element_type=f32).astype(jnp.bfloat16)
        wmsg_ref[1, :_D] = jnp.dot(
            ga, wout_ref[...],
            preferred_element_type=f32).astype(jnp.bfloat16)
        wmsg_ref[1, _D:] = jnp.dot(
            gb, wout_ref[...],
            preferred_element_type=f32).astype(jnp.bfloat16)
        bl2 = jnp.dot(lr_ref[...], dft2_ref[...], preferred_element_type=f32)
        gaw = jnp.dot(ga, wl_ref[...], preferred_element_type=f32)
        gbw = jnp.dot(gb, wl_ref[...], preferred_element_type=f32)
        wle_ref[...] = (bl2[0, :_D, None] * gaw + bl2[0, _D:, None] * gbw)
        relout_ref[...] = jnp.dot(rel_ref[...], wr_ref[...],
                                  preferred_element_type=f32)


def _tc_prep(x, rel, loop_rel, w_in, w_out, w_loop, w_rel, dft2, ga, gb):
    zero2 = lambda i: (0, 0)
    zero3 = lambda i: (0, 0, 0)
    return pl.pallas_call(
        _prep_body,
        grid=(_N // _NC_CH,),
        in_specs=[
            pl.BlockSpec((_NC_CH, _D), lambda i: (i, 0)),
            pl.BlockSpec((_R2, _D), zero2),
            pl.BlockSpec((1, _D), zero2),
            pl.BlockSpec((_D, _D), zero2),
            pl.BlockSpec((_D, _D), zero2),
            pl.BlockSpec((_D, _D), zero2),
            pl.BlockSpec((_D, _D), zero2),
            pl.BlockSpec((_D, 2 * _D), zero2),
            pl.BlockSpec((_D, _D), zero2),
            pl.BlockSpec((_D, _D), zero2),
        ],
        out_specs=[
            pl.BlockSpec((_NC_CH, _D), lambda i: (i, 0)),
            pl.BlockSpec((_R2, 2 * _D), zero2),
            pl.BlockSpec((2, 2 * _D, _D), zero3),
            pl.BlockSpec((_D, _D), zero2),
            pl.BlockSpec((_R2, _D), zero2),
        ],
        out_shape=[
            jax.ShapeDtypeStruct((_N, _D), jnp.float32),
            jax.ShapeDtypeStruct((_R2, 2 * _D), jnp.bfloat16),
            jax.ShapeDtypeStruct((2, 2 * _D, _D), jnp.bfloat16),
            jax.ShapeDtypeStruct((_D, _D), jnp.float32),
            jax.ShapeDtypeStruct((_R2, _D), jnp.float32),
        ],
    )(x, rel, loop_rel, w_in, w_out, w_loop, w_rel, dft2, ga, gb)


def _sc_gather(table, idx2d):
    mesh = plsc.VectorSubcoreMesh(core_axis_name="c", subcore_axis_name="s")
    nc, ns = mesh.num_cores, mesh.num_subcores
    per_w = _E // (nc * ns)
    iters = per_w // _SC_CH

    assert iters % 2 == 1 and iters >= 3
    pairs = (iters - 1) // 2

    @functools.partial(
        pl.kernel,
        out_type=jax.ShapeDtypeStruct((_E, _D), jnp.float32),
        mesh=mesh,
        scratch_types=[
            pltpu.VMEM((iters, _SC_CH), jnp.int32),
            pltpu.VMEM((_SC_CH, _D), jnp.float32),
            pltpu.VMEM((_SC_CH, _D), jnp.float32),
            pltpu.SemaphoreType.DMA,
            pltpu.SemaphoreType.DMA,
        ],
    )
    def gk(table_hbm, idx_hbm, out_hbm, idx_v, rows0, rows1, sem0, sem1):
        c = lax.axis_index("c")
        s = lax.axis_index("s")
        wid = s * nc + c
        base = wid * per_w
        pltpu.sync_copy(idx_hbm.at[wid], idx_v)

        def fire(i, rows, sem):
            pltpu.async_copy(table_hbm.at[idx_v.at[i]], rows, sem)

        def drain(i, rows, sem):
            pltpu.make_async_copy(table_hbm.at[idx_v.at[i]], rows,
                                  sem).wait()
            pltpu.sync_copy(rows,
                            out_hbm.at[pl.ds(base + i * _SC_CH, _SC_CH)])

        fire(0, rows0, sem0)

        def body(j, carry):
            fire(2 * j + 1, rows1, sem1)
            drain(2 * j, rows0, sem0)
            fire(2 * j + 2, rows0, sem0)
            drain(2 * j + 1, rows1, sem1)
            return carry

        lax.fori_loop(0, pairs, body, 0)
        drain(iters - 1, rows0, sem0)

    return gk(table, idx2d)


def _msg_body(z_ref, et_ref, nrm_ref, frel_ref, w_ref, o_ref):
    f32 = jnp.float32
    et_row = et_ref[0]
    nrm_row = nrm_ref[0]
    iota_r = lax.broadcasted_iota(jnp.int32, (_R2, _EC), 0)
    et_b = jnp.broadcast_to(et_row, (_R2, _EC))
    nrm_b = jnp.broadcast_to(nrm_row, (_R2, _EC))
    oh_t = jnp.where(et_b == iota_r, nrm_b,
                     jnp.float32(0)).astype(jnp.bfloat16)
    b2 = lax.dot_general(oh_t, frel_ref[...],
                         (((0,), (0,)), ((), ())),
                         preferred_element_type=f32)
    a = z_ref[...]
    p = (a * b2[:, :_D]).astype(jnp.bfloat16)
    q = (a * b2[:, _D:]).astype(jnp.bfloat16)
    pq = jnp.concatenate([p, q], axis=1)
    o_ref[...] = jnp.dot(pq, w_ref[0], preferred_element_type=f32)


def _tc_msg(zsrc, edge_type3, norm3, frel2, wmsg):
    nblk = _E // _EC
    half_blk = nblk // 2
    return pl.pallas_call(
        _msg_body,
        grid=(nblk,),
        in_specs=[
            pl.BlockSpec((_EC, _D), lambda i: (i, 0)),
            pl.BlockSpec((1, 1, _EC), lambda i: (i, 0, 0)),
            pl.BlockSpec((1, 1, _EC), lambda i: (i, 0, 0)),
            pl.BlockSpec((_R2, 2 * _D), lambda i: (0, 0)),
            pl.BlockSpec((1, 2 * _D, _D), lambda i: (i // half_blk, 0, 0)),
        ],
        out_specs=pl.BlockSpec((_EC, _D), lambda i: (i, 0)),
        out_shape=jax.ShapeDtypeStruct((_E, _D), jnp.float32),
    )(zsrc, edge_type3, norm3, frel2, wmsg)


def _sc_scatter(msg, dst):
    mesh = plsc.VectorSubcoreMesh(core_axis_name="c", subcore_axis_name="s")
    nc, ns = mesh.num_cores, mesh.num_subcores
    per_w = _E // (nc * ns)
    iters = per_w // _SC_CH
    rows_s = _NPAD // ns
    zreps = rows_s // _SC_CH

    assert iters % 2 == 1 and iters >= 3
    pairs = (iters - 1) // 2

    @functools.partial(
        pl.kernel,
        out_type=jax.ShapeDtypeStruct((nc, _NPAD, _D), jnp.float32),
        mesh=mesh,
        scratch_types=[
            pltpu.VMEM((iters, _SC_CH), jnp.int32),
            pltpu.VMEM((_SC_CH, _D), jnp.float32),
            pltpu.VMEM((_SC_CH, _D), jnp.float32),
            pltpu.VMEM_SHARED((_NPAD, _D), jnp.float32),
            pltpu.SemaphoreType.DMA,
            pltpu.SemaphoreType.DMA,
        ],
    )
    def sk(msg_hbm, dst_hbm, out_hbm, idx_v, rows0, rows1, acc_sh,
           sem0, sem1):
        c = lax.axis_index("c")
        s = lax.axis_index("s")
        wid = s * nc + c
        base = wid * per_w
        pltpu.sync_copy(dst_hbm.at[wid], idx_v)

        def zb(i, carry):
            rows0[i // 8, pl.ds((i % 8) * 16, 16)] = jnp.zeros((16,),
                                                               jnp.float32)
            return carry

        lax.fori_loop(0, _SC_CH * 8, zb, 0)

        def zs(j, carry):
            pltpu.sync_copy(
                rows0,
                acc_sh.at[pl.ds(s * rows_s + j * _SC_CH, _SC_CH)])
            return carry

        lax.fori_loop(0, zreps, zs, 0)
        plsc.subcore_barrier()

        def fire(i, rows, sem):
            pltpu.async_copy(msg_hbm.at[pl.ds(base + i * _SC_CH, _SC_CH)],
                             rows, sem)

        def drain(i, rows, sem):
            pltpu.make_async_copy(msg_hbm.at[pl.ds(base + i * _SC_CH,
                                                   _SC_CH)], rows,
                                  sem).wait()
            pltpu.sync_copy(rows, acc_sh.at[idx_v.at[i]], add=True)

        fire(0, rows0, sem0)

        def body(j, carry):
            fire(2 * j + 1, rows1, sem1)
            drain(2 * j, rows0, sem0)
            fire(2 * j + 2, rows0, sem0)
            drain(2 * j + 1, rows1, sem1)
            return carry

        lax.fori_loop(0, pairs, body, 0)
        drain(iters - 1, rows0, sem0)
        plsc.subcore_barrier()
        pltpu.sync_copy(acc_sh.at[pl.ds(s * rows_s, rows_s)],
                        out_hbm.at[c, pl.ds(s * rows_s, rows_s)])

    return sk(msg, dst)


_NB = _N // _NC_CH


def _fin_body(p_ref, fx_ref, wl_ref, b_ref, g_ref, bt_ref, o_ref,
              t_ref, st_ref):
    i = pl.program_id(0)

    @pl.when(i < _NB)
    def _():
        lm = jnp.dot(fx_ref[...], wl_ref[...],
                     preferred_element_type=jnp.float32)
        tt = ((p_ref[0] + p_ref[1] + lm) * jnp.float32(1.0 / 3.0)
              + b_ref[...])
        blk = lax.rem(i, _NB)
        t_ref[pl.ds(blk * _NC_CH, _NC_CH), :] = tt
        s1 = jnp.sum(tt, axis=0, keepdims=True)
        s2 = jnp.sum(tt * tt, axis=0, keepdims=True)
        upd = jnp.concatenate([s1, s2, jnp.zeros((6, _D), jnp.float32)],
                              axis=0)

        @pl.when(i == 0)
        def _():
            st_ref[...] = jnp.zeros((8, _D), jnp.float32)

        st_ref[...] += upd

    @pl.when(i >= _NB)
    def _():
        inv_n = jnp.float32(1.0 / _N)
        mean = st_ref[0:1] * inv_n
        var = st_ref[1:2] * inv_n - mean * mean
        inv = lax.rsqrt(var + jnp.float32(1e-5))
        blk = lax.rem(i, _NB)
        tt = t_ref[pl.ds(blk * _NC_CH, _NC_CH), :]
        o_ref[...] = jnp.tanh((tt - mean) * inv * g_ref[...] + bt_ref[...])


def _tc_finish(partials, fx, wloop_eff, bias, gamma, beta):
    mod = lambda i: (lax.rem(i, _NB), 0)
    zero2 = lambda i: (0, 0)
    return pl.pallas_call(
        _fin_body,
        grid=(2 * _NB,),
        in_specs=[
            pl.BlockSpec((2, _NC_CH, _D), lambda i: (0, lax.rem(i, _NB), 0)),
            pl.BlockSpec((_NC_CH, _D), mod),
            pl.BlockSpec((_D, _D), zero2),
            pl.BlockSpec((1, _D), zero2),
            pl.BlockSpec((1, _D), zero2),
            pl.BlockSpec((1, _D), zero2),
        ],
        out_specs=pl.BlockSpec((_NC_CH, _D), mod),
        out_shape=jax.ShapeDtypeStruct((_N, _D), jnp.float32),
        scratch_shapes=[
            pltpu.VMEM((_N, _D), jnp.float32),
            pltpu.VMEM((8, _D), jnp.float32),
        ],
    )(partials, fx, wloop_eff, bias, gamma, beta)


def kernel(x, edge_index, edge_type, norm, rel_embeds, w_loop, w_in, w_out,
           w_rel, loop_rel, w_bias, bn_gamma, bn_beta):
    src = edge_index[0]
    dst = edge_index[1]
    dft2 = jnp.asarray(_DFT2)
    dftc = dft2[:, :_D]
    ga = jnp.asarray(_GA)
    gb = jnp.asarray(_GB)

    fx, frel2, wmsg, wloop_eff, rel_out = _tc_prep(
        x, rel_embeds, loop_rel, w_in, w_out, w_loop, w_rel, dft2, ga, gb)
    nw = _E // (_SC_CH * 125)
    zsrc = _sc_gather(fx, src.reshape(nw, 125, _SC_CH))
    nblk = _E // _EC
    msg = _tc_msg(zsrc, edge_type.reshape(nblk, 1, _EC),
                  norm.reshape(nblk, 1, _EC), frel2, wmsg)
    partials = _sc_scatter(msg, dst.reshape(nw, 125, _SC_CH))
    node_repr = _tc_finish(partials, fx, wloop_eff, w_bias.reshape(1, _D),
                           bn_gamma.reshape(1, _D), bn_beta.reshape(1, _D))
    return node_repr, rel_out
